# baseline (speedup 1.0000x reference)
import functools

import jax
import jax.numpy as jnp
import numpy as np
from jax.experimental import pallas as pl
from jax.experimental.pallas import tpu as pltpu

_OBJECT_PROBS = (0.1, 0.1, 0.1, 0.1, 0.15, 0.15, 0.05, 0.05)
_COLS_PER_OBJ = 32
_MASK_VALUE = 0.0


def _threefry2x32_pair(keypair, x0, x1):
    def rotl(v, d):
        return ((v << np.uint32(d)) | (v >> np.uint32(32 - d))).astype(np.uint32)

    x = [np.asarray(x0, np.uint32).copy(), np.asarray(x1, np.uint32).copy()]
    rotations = ((13, 15, 26, 6), (17, 29, 16, 24))
    k0, k1 = np.uint32(keypair[0]), np.uint32(keypair[1])
    ks = [k0, k1, k0 ^ k1 ^ np.uint32(0x1BD11BDA)]
    x[0] = (x[0] + ks[0]).astype(np.uint32)
    x[1] = (x[1] + ks[1]).astype(np.uint32)
    for i in range(5):
        for r in rotations[i % 2]:
            x[0] = (x[0] + x[1]).astype(np.uint32)
            x[1] = rotl(x[1], r)
            x[1] = x[1] ^ x[0]
        x[0] = (x[0] + ks[(i + 1) % 3]).astype(np.uint32)
        x[1] = (x[1] + ks[(i + 2) % 3] + np.uint32(i + 1)).astype(np.uint32)
    return x


def _fold_in(keypair, i):
    o = _threefry2x32_pair(keypair, np.array([0], np.uint32), np.array([i], np.uint32))
    return np.uint32(o[0][0]), np.uint32(o[1][0])


def _np_uniform(keypair, n):
    idx = np.arange(n, dtype=np.uint64)
    o = _threefry2x32_pair(keypair, (idx >> np.uint64(32)).astype(np.uint32),
                           idx.astype(np.uint32))
    bits = o[0] ^ o[1]
    return ((bits >> np.uint32(9)) | np.uint32(0x3F800000)).view(np.float32) - np.float32(1.0)


@functools.lru_cache(maxsize=None)
def _mask_bits(batch):
    root = (np.uint32(0), np.uint32(42))
    bits = np.zeros((batch,), np.int32)
    for i, p in enumerate(_OBJECT_PROBS):
        m = _np_uniform(_fold_in(root, i), batch) < np.float32(p)
        bits |= m.astype(np.int32) << i
    return bits.reshape(batch, 1)


def _mask_kernel(bits_ref, x_ref, o_ref):
    x = x_ref[...]
    bits = bits_ref[...]
    obj = jax.lax.broadcasted_iota(jnp.int32, x.shape, 1) // _COLS_PER_OBJ
    masked = (jnp.right_shift(bits, obj) & 1) != 0
    o_ref[...] = jnp.where(masked, jnp.float32(_MASK_VALUE), x)


def _tc_mask_rows(x, r1, rows):
    b, f = x.shape
    bits = jnp.asarray(_mask_bits(b))
    return pl.pallas_call(
        _mask_kernel,
        grid=(r1 // rows,),
        in_specs=[
            pl.BlockSpec((rows, 1), lambda i: (i, 0)),
            pl.BlockSpec((rows, f), lambda i: (i, 0)),
        ],
        out_specs=pl.BlockSpec((rows, f), lambda i: (i, 0)),
        out_shape=jax.ShapeDtypeStruct((r1, f), x.dtype),
        compiler_params=pltpu.CompilerParams(
            dimension_semantics=("parallel",),
        ),
    )(bits, x)


_NW = 32
_CHUNK_ROWS = 128
_LANES = 16


@functools.lru_cache(maxsize=None)
def _sc_consts(batch, n_feat, r0, r1):
    n_obj = len(_OBJECT_PROBS)
    seg_w = n_feat // n_obj
    bits = _mask_bits(batch).ravel()[r0:r1]
    rows, objs = np.nonzero((bits[:, None] >> np.arange(n_obj)) & 1)
    rows = rows.astype(np.int32)
    cols = (objs * seg_w).astype(np.int32)
    rows_per_w = (r1 - r0) // _NW
    nch = rows_per_w // _CHUNK_ROWS
    lists = [[None] * nch for _ in range(_NW)]
    for w in range(_NW):
        for t in range(nch):
            lo = w * rows_per_w + t * _CHUNK_ROWS
            sel = (rows >= lo) & (rows < lo + _CHUNK_ROWS)
            assert sel.any()
            lists[w][t] = (rows[sel] - lo, cols[sel])
    g_max = max(-(-len(l[0]) // _LANES) for row in lists for l in row)
    ridx = np.empty((_NW, nch * g_max * _LANES), np.int32)
    cidx = np.empty((_NW, nch * g_max * _LANES), np.int32)
    for w in range(_NW):
        for t in range(nch):
            r, c = lists[w][t]
            sl = slice(t * g_max * _LANES, (t + 1) * g_max * _LANES)
            rp = np.full(g_max * _LANES, r[0], np.int32)
            cp = np.full(g_max * _LANES, c[0], np.int32)
            rp[: len(r)] = r
            cp[: len(c)] = c
            ridx[w, sl] = rp
            cidx[w, sl] = cp
    return ridx, cidx, nch, g_max


def _sc_mask_rows(x, r0, r1):
    b, f = x.shape
    n_obj = len(_OBJECT_PROBS)
    seg_w = f // n_obj
    rows_per_w = (r1 - r0) // _NW
    ridx_np, cidx_np, nch, g_max = _sc_consts(b, f, r0, r1)

    from jax.experimental.pallas import tpu_sc as plsc

    mesh = plsc.VectorSubcoreMesh(core_axis_name="c", subcore_axis_name="s")
    n_cores = mesh.num_cores

    @functools.partial(
        pl.kernel,
        out_type=jax.ShapeDtypeStruct((r1 - r0, f), jnp.float32),
        mesh=mesh,
        scratch_types=[
            pltpu.VMEM((_CHUNK_ROWS, f), jnp.float32),
            pltpu.VMEM((_CHUNK_ROWS, f), jnp.float32),
            pltpu.VMEM((ridx_np.shape[1],), jnp.int32),
            pltpu.VMEM((cidx_np.shape[1],), jnp.int32),
            pltpu.SemaphoreType.DMA,
            pltpu.SemaphoreType.DMA,
            pltpu.SemaphoreType.DMA,
            pltpu.SemaphoreType.DMA,
        ],
        compiler_params=pltpu.CompilerParams(needs_layout_passes=False),
    )
    def sc_kernel(x_hbm, ridx_hbm, cidx_hbm, out_hbm,
                  buf0, buf1, rv, cv, l0, l1, s0, s1):
        wid = jax.lax.axis_index("s") * n_cores + jax.lax.axis_index("c")
        base = r0 + wid * rows_per_w
        obase = wid * rows_per_w
        bufs, lsems, ssems = (buf0, buf1), (l0, l1), (s0, s1)
        pltpu.sync_copy(ridx_hbm.at[wid], rv)
        pltpu.sync_copy(cidx_hbm.at[wid], cv)
        zeros = jnp.zeros((_LANES,), jnp.float32)

        def load(t):
            return pltpu.make_async_copy(
                x_hbm.at[pl.ds(base + t * _CHUNK_ROWS, _CHUNK_ROWS)],
                bufs[t % 2], lsems[t % 2])

        def store(t):
            return pltpu.make_async_copy(
                bufs[t % 2],
                out_hbm.at[pl.ds(obase + t * _CHUNK_ROWS, _CHUNK_ROWS)],
                ssems[t % 2])

        load(0).start()
        for t in range(nch):
            if t + 1 < nch:
                if t >= 1:
                    store(t - 1).wait()
                load(t + 1).start()
            load(t).wait()
            for g in range(g_max):
                o = (t * g_max + g) * _LANES
                a = rv[pl.ds(o, _LANES)]
                cbase = cv[pl.ds(o, _LANES)]
                for c in range(seg_w):
                    plsc.store_scatter(bufs[t % 2], [a, cbase + c], zeros)
            store(t).start()
        if nch >= 2:
            store(nch - 2).wait()
        store(nch - 1).wait()

    return sc_kernel(x, jnp.asarray(ridx_np), jnp.asarray(cidx_np))


def kernel(x):
    b, _ = x.shape
    split = (3 * b) // 4
    top = _tc_mask_rows(x, split, rows=4096)
    bot = _sc_mask_rows(x, split, b)
    return jnp.concatenate([top, bot], axis=0)


# speedup vs baseline: 3.2064x; 3.2064x over previous
import functools

import jax
import jax.numpy as jnp
import numpy as np
from jax.experimental import pallas as pl
from jax.experimental.pallas import tpu as pltpu

_OBJECT_PROBS = (0.1, 0.1, 0.1, 0.1, 0.15, 0.15, 0.05, 0.05)
_COLS_PER_OBJ = 32
_MASK_VALUE = 0.0


def _threefry2x32_pair(keypair, x0, x1):
    def rotl(v, d):
        return ((v << np.uint32(d)) | (v >> np.uint32(32 - d))).astype(np.uint32)

    x = [np.asarray(x0, np.uint32).copy(), np.asarray(x1, np.uint32).copy()]
    rotations = ((13, 15, 26, 6), (17, 29, 16, 24))
    k0, k1 = np.uint32(keypair[0]), np.uint32(keypair[1])
    ks = [k0, k1, k0 ^ k1 ^ np.uint32(0x1BD11BDA)]
    x[0] = (x[0] + ks[0]).astype(np.uint32)
    x[1] = (x[1] + ks[1]).astype(np.uint32)
    for i in range(5):
        for r in rotations[i % 2]:
            x[0] = (x[0] + x[1]).astype(np.uint32)
            x[1] = rotl(x[1], r)
            x[1] = x[1] ^ x[0]
        x[0] = (x[0] + ks[(i + 1) % 3]).astype(np.uint32)
        x[1] = (x[1] + ks[(i + 2) % 3] + np.uint32(i + 1)).astype(np.uint32)
    return x


def _fold_in(keypair, i):
    o = _threefry2x32_pair(keypair, np.array([0], np.uint32), np.array([i], np.uint32))
    return np.uint32(o[0][0]), np.uint32(o[1][0])


def _np_uniform(keypair, n):
    idx = np.arange(n, dtype=np.uint64)
    o = _threefry2x32_pair(keypair, (idx >> np.uint64(32)).astype(np.uint32),
                           idx.astype(np.uint32))
    bits = o[0] ^ o[1]
    return ((bits >> np.uint32(9)) | np.uint32(0x3F800000)).view(np.float32) - np.float32(1.0)


@functools.lru_cache(maxsize=None)
def _mask_bits(batch):
    root = (np.uint32(0), np.uint32(42))
    bits = np.zeros((batch,), np.int32)
    for i, p in enumerate(_OBJECT_PROBS):
        m = _np_uniform(_fold_in(root, i), batch) < np.float32(p)
        bits |= m.astype(np.int32) << i
    return bits.reshape(batch, 1)


def _mask_kernel(bits_ref, x_ref, o_ref):
    x = x_ref[...]
    bits = bits_ref[...]
    obj = jax.lax.broadcasted_iota(jnp.int32, x.shape, 1) // _COLS_PER_OBJ
    masked = (jnp.right_shift(bits, obj) & 1) != 0
    o_ref[...] = jnp.where(masked, jnp.float32(_MASK_VALUE), x)


def kernel(x):
    b, f = x.shape
    bits = jnp.asarray(_mask_bits(b))
    rows = 8192
    return pl.pallas_call(
        _mask_kernel,
        grid=(b // rows,),
        in_specs=[
            pl.BlockSpec((rows, 1), lambda i: (i, 0)),
            pl.BlockSpec((rows, f), lambda i: (i, 0)),
        ],
        out_specs=pl.BlockSpec((rows, f), lambda i: (i, 0)),
        out_shape=jax.ShapeDtypeStruct((b, f), x.dtype),
        compiler_params=pltpu.CompilerParams(
            dimension_semantics=("parallel",),
        ),
    )(bits, x)
